# Initial kernel scaffold; baseline (speedup 1.0000x reference)
import jax
import jax.numpy as jnp
from jax.experimental import pallas as pl


def kernel(x, W):
    raise NotImplementedError("write your pallas kernel here")


# speedup vs baseline: 4.7398x; 4.7398x over previous
import functools

import jax
import jax.numpy as jnp
from jax import lax
from jax.experimental import pallas as pl
from jax.experimental.pallas import tpu as pltpu
from jax.experimental.pallas import tpu_sc as plsc

_NC = 2
_NS = 16
_NW = _NC * _NS

_SUB = 128
_NSUB = 4
_CHUNK = _SUB * _NSUB


@functools.partial(jax.jit, static_argnums=(2, 3))
def _gather(idx2d, table, B, D):
    b_per_w = B // _NW
    n_chunks = b_per_w // _CHUNK
    idxrows_per_w = b_per_w // _SUB

    mesh = plsc.VectorSubcoreMesh(
        core_axis_name="c", subcore_axis_name="s",
        num_cores=_NC, num_subcores=_NS)

    @functools.partial(
        pl.kernel,
        out_type=jax.ShapeDtypeStruct((B, D), jnp.float32),
        mesh=mesh,
        scratch_types=[
            pltpu.VMEM((_NSUB, _SUB), jnp.int32),
            pltpu.VMEM((_CHUNK, D), jnp.float32),
            pltpu.SemaphoreType.DMA,
        ],
        compiler_params=pltpu.CompilerParams(use_tc_tiling_on_sc=False),
    )
    def k(idx_hbm, table_hbm, out_hbm, idx_v, rows_v, sem):
        wid = lax.axis_index("s") * _NC + lax.axis_index("c")
        idx_row0 = wid * idxrows_per_w
        out_row0 = wid * b_per_w

        @pl.loop(0, n_chunks)
        def _chunk(g):
            pltpu.sync_copy(
                idx_hbm.at[pl.ds(idx_row0 + g * _NSUB, _NSUB)], idx_v)
            waits = []
            for j in range(_NSUB):
                waits.append(pltpu.async_copy(
                    table_hbm.at[idx_v.at[j]],
                    rows_v.at[pl.ds(j * _SUB, _SUB)],
                    sem))
            for w in waits:
                w.wait()
            pltpu.sync_copy(
                rows_v, out_hbm.at[pl.ds(out_row0 + g * _CHUNK, _CHUNK)])

    return k(idx2d, table)


def kernel(x, W):
    B = x.shape[0] * x.shape[1]
    D = W.shape[1]
    idx2d = x.reshape(B // _SUB, _SUB).astype(jnp.int32)
    out = _gather(idx2d, W, B, D)
    return out.reshape(x.shape[0], x.shape[1], D)


# speedup vs baseline: 5.1718x; 1.0911x over previous
import functools

import jax
import jax.numpy as jnp
from jax import lax
from jax.experimental import pallas as pl
from jax.experimental.pallas import tpu as pltpu
from jax.experimental.pallas import tpu_sc as plsc

_NC = 2
_NS = 16
_NW = _NC * _NS

_SUB = 128
_NSUB = 5
_CHUNK = _SUB * _NSUB
_NBUF = 2


@functools.partial(jax.jit, static_argnums=(2, 3))
def _gather(idx2d, table, B, D):
    b_per_w = B // _NW
    n_chunks = b_per_w // _CHUNK
    npair = n_chunks // _NBUF
    idxrows_per_w = b_per_w // _SUB

    mesh = plsc.VectorSubcoreMesh(
        core_axis_name="c", subcore_axis_name="s",
        num_cores=_NC, num_subcores=_NS)

    @functools.partial(
        pl.kernel,
        out_type=jax.ShapeDtypeStruct((B, D), jnp.float32),
        mesh=mesh,
        scratch_types=[
            pltpu.VMEM((_NBUF, _NSUB, _SUB), jnp.int32),
            pltpu.VMEM((_NBUF, _CHUNK, D), jnp.float32),
            [pltpu.SemaphoreType.DMA] * _NBUF,
            [pltpu.SemaphoreType.DMA] * _NBUF,
            [pltpu.SemaphoreType.DMA] * _NBUF,
        ],
        compiler_params=pltpu.CompilerParams(use_tc_tiling_on_sc=False),
    )
    def k(idx_hbm, table_hbm, out_hbm, idx_v, rows_v, semi, semg, semo):
        wid = lax.axis_index("s") * _NC + lax.axis_index("c")
        idx_row0 = wid * idxrows_per_w
        out_row0 = wid * b_per_w

        def start_idx(b, g):
            pltpu.async_copy(
                idx_hbm.at[pl.ds(idx_row0 + g * _NSUB, _NSUB)],
                idx_v.at[b], semi[b])

        def wait_idx(b):
            pltpu.make_async_copy(
                idx_hbm.at[pl.ds(idx_row0, _NSUB)],
                idx_v.at[b], semi[b]).wait()

        def run_gather(b):
            waits = []
            for j in range(_NSUB):
                waits.append(pltpu.async_copy(
                    table_hbm.at[idx_v.at[b, j]],
                    rows_v.at[b, pl.ds(j * _SUB, _SUB)],
                    semg[b]))
            for w in waits:
                w.wait()

        def start_out(b, g):
            pltpu.async_copy(
                rows_v.at[b],
                out_hbm.at[pl.ds(out_row0 + g * _CHUNK, _CHUNK)],
                semo[b])

        def wait_out(b):
            pltpu.make_async_copy(
                rows_v.at[b],
                out_hbm.at[pl.ds(out_row0, _CHUNK)],
                semo[b]).wait()

        for b in range(_NBUF):
            start_idx(b, b)
        for b in range(_NBUF):
            wait_idx(b)
            run_gather(b)
            start_idx(b, b + _NBUF)
            start_out(b, b)

        @pl.loop(1, npair - 1)
        def _pair(p):
            for b in range(_NBUF):
                g = p * _NBUF + b
                wait_idx(b)
                wait_out(b)
                run_gather(b)
                start_idx(b, g + _NBUF)
                start_out(b, g)

        for b in range(_NBUF):
            g = n_chunks - _NBUF + b
            wait_idx(b)
            wait_out(b)
            run_gather(b)
            start_out(b, g)
        for b in range(_NBUF):
            wait_out(b)

    return k(idx2d, table)


def kernel(x, W):
    B = x.shape[0] * x.shape[1]
    D = W.shape[1]
    idx2d = x.reshape(B // _SUB, _SUB).astype(jnp.int32)
    out = _gather(idx2d, W, B, D)
    return out.reshape(x.shape[0], x.shape[1], D)
